# baseline (speedup 1.0000x reference)
import functools

import jax
import jax.numpy as jnp
from jax import lax
from jax.experimental import pallas as pl
from jax.experimental.pallas import tpu as pltpu
from jax.experimental.pallas import tpu_sc as plsc

_L = 16
_EPS = 1e-12


def _shuffle(v, idx):
    return lax.gather(
        v, idx[:, None],
        dimension_numbers=lax.GatherDimensionNumbers(
            offset_dims=(), collapsed_slice_dims=(0,), start_index_map=(0,)),
        slice_sizes=(1,),
        mode=lax.GatherScatterMode.PROMISE_IN_BOUNDS)


def _lane_total(v):
    lane = lax.iota(jnp.int32, _L)
    for k in (8, 4, 2, 1):
        v = v + _shuffle(v, lane ^ k)
    return v


def _rsqrt(v):
    i = lax.bitcast_convert_type(v, jnp.int32)
    i = jnp.int32(0x5F3759DF) - lax.shift_right_logical(i, 1)
    y = lax.bitcast_convert_type(i, jnp.float32)
    for _ in range(3):
        y = y * (1.5 - 0.5 * v * y * y)
    return y


def kernel(inputs, token_table, pos_table, gamma, beta):
    B, S = inputs.shape
    V, H = token_table.shape
    nj = H // _L

    info = plsc.get_sparse_core_info()
    NC, NS = info.num_cores, info.num_subcores
    NW = NC * NS
    assert B % NW == 0 and S % 2 == 0 and H % _L == 0
    seq_per_w = B // NW
    half = S // 2

    idx2 = inputs.reshape(B, 2, half)
    mesh = plsc.VectorSubcoreMesh(core_axis_name="c", subcore_axis_name="s")

    @functools.partial(
        pl.kernel,
        mesh=mesh,
        out_type=jax.ShapeDtypeStruct((B, S, H), jnp.float32),
        compiler_params=pltpu.CompilerParams(use_tc_tiling_on_sc=False),
        scratch_types=[
            pltpu.VMEM((2, half), jnp.int32),
            pltpu.VMEM((S, H), jnp.float32),
            pltpu.VMEM((S, H), jnp.float32),
            pltpu.VMEM((H,), jnp.float32),
            pltpu.VMEM((H,), jnp.float32),
            pltpu.SemaphoreType.DMA,
        ],
    )
    def sc_kernel(idx_hbm, tok_hbm, pos_hbm, gamma_hbm, beta_hbm, out_hbm,
                  idx_v, buf, pos_v, g_v, b_v, sem):
        wid = lax.axis_index("s") * NC + lax.axis_index("c")

        pltpu.sync_copy(pos_hbm.at[pl.ds(0, S)], pos_v)
        pltpu.sync_copy(gamma_hbm, g_v)
        pltpu.sync_copy(beta_hbm, b_v)
        g = [g_v[pl.ds(_L * j, _L)] for j in range(nj)]
        bt = [b_v[pl.ds(_L * j, _L)] for j in range(nj)]

        def seq_body(i, carry):
            b = wid * seq_per_w + i
            pltpu.sync_copy(idx_hbm.at[b], idx_v)
            cp0 = pltpu.async_copy(tok_hbm.at[idx_v.at[0]],
                                   buf.at[pl.ds(0, half)], sem)
            cp1 = pltpu.async_copy(tok_hbm.at[idx_v.at[1]],
                                   buf.at[pl.ds(half, half)], sem)
            cp0.wait()
            cp1.wait()

            def row_body(s, c):
                x = [buf[s, pl.ds(_L * j, _L)] + pos_v[s, pl.ds(_L * j, _L)]
                     for j in range(nj)]
                tot = x[0]
                for j in range(1, nj):
                    tot = tot + x[j]
                mean = _lane_total(tot) * (1.0 / H)
                sq = x[0] * x[0]
                for j in range(1, nj):
                    sq = sq + x[j] * x[j]
                var = _lane_total(sq) * (1.0 / H) - mean * mean
                rstd = _rsqrt(var + _EPS)
                for j in range(nj):
                    buf[s, pl.ds(_L * j, _L)] = (x[j] - mean) * rstd * g[j] + bt[j]
                return c

            lax.fori_loop(0, S, row_body, 0)
            pltpu.sync_copy(buf, out_hbm.at[b])
            return carry

        lax.fori_loop(0, seq_per_w, seq_body, 0)

    return sc_kernel(idx2, token_table, pos_table, gamma, beta)
